# baseline (speedup 1.0000x reference)
import functools

import jax
import jax.numpy as jnp
from jax.experimental import pallas as pl
from jax.experimental.pallas import tpu as pltpu

BATCH = 4
SEQ_LEN = 8192
D_MODEL = 4096
N_EXPERTS = 64
EXPERT_CAPACITY = 160

BLK_S = 256


def _router_block(x_ref, w_ref, b_ref, ei_ref, tp_ref, rp_ref, aux_ref,
                  carry_ref, fi_ref, pi_ref):
    b = pl.program_id(0)
    i = pl.program_id(1)
    nblk = pl.num_programs(1)

    @pl.when(i == 0)
    def _reset():
        carry_ref[...] = jnp.zeros_like(carry_ref)
        fi_ref[...] = jnp.zeros_like(fi_ref)
        pi_ref[...] = jnp.zeros_like(pi_ref)

    x = x_ref[0]
    logits = jnp.dot(x, w_ref[...],
                     preferred_element_type=jnp.float32) + b_ref[...]
    m = jnp.max(logits, axis=-1, keepdims=True)
    e = jnp.exp(logits - m)
    s = jnp.sum(e, axis=-1, keepdims=True)
    probs = e / s
    rp_ref[0] = probs

    maxp = jnp.max(probs, axis=-1, keepdims=True)
    tp_ref[0] = maxp

    lane = jax.lax.broadcasted_iota(jnp.int32, probs.shape, 1)
    cand = jnp.where(probs >= maxp, lane, N_EXPERTS)
    top_idx = jnp.min(cand, axis=-1, keepdims=True)
    onehot_f = (lane == top_idx).astype(jnp.float32)

    row = jax.lax.broadcasted_iota(jnp.int32, (BLK_S, BLK_S), 0)
    col = jax.lax.broadcasted_iota(jnp.int32, (BLK_S, BLK_S), 1)
    tri = (row >= col).astype(jnp.float32)
    prio_local = jax.lax.dot_general(
        tri, onehot_f, (((1,), (0,)), ((), ())),
        preferred_element_type=jnp.float32)
    prio = prio_local + carry_ref[...]
    keep = prio <= EXPERT_CAPACITY
    kept = jnp.where(keep, onehot_f, 0.0)
    ei_ref[0] = kept.astype(jnp.int32)

    carry_ref[...] = prio[BLK_S - 1:BLK_S, :]
    fi_ref[...] += jnp.sum(kept, axis=0, keepdims=True)
    pi_ref[...] += jnp.sum(probs, axis=0, keepdims=True)

    @pl.when(i == nblk - 1)
    def _aux():
        partial = (N_EXPERTS / (BATCH * float(SEQ_LEN) * float(SEQ_LEN))) * \
            jnp.sum(fi_ref[...] * pi_ref[...])

        @pl.when(b == 0)
        def _init():
            aux_ref[...] = jnp.full((1, 1), partial, jnp.float32)

        @pl.when(b != 0)
        def _acc():
            aux_ref[...] += partial


@jax.jit
def kernel(hidden_states, W, b):
    B, S, D = hidden_states.shape
    E = W.shape[1]
    nblk = S // BLK_S
    grid = (B, nblk)

    ei, tp, rp, aux = pl.pallas_call(
        _router_block,
        grid=grid,
        in_specs=[
            pl.BlockSpec((1, BLK_S, D), lambda b_, i: (b_, i, 0)),
            pl.BlockSpec((D, E), lambda b_, i: (0, 0)),
            pl.BlockSpec((1, E), lambda b_, i: (0, 0)),
        ],
        out_specs=[
            pl.BlockSpec((1, BLK_S, E), lambda b_, i: (b_, i, 0)),
            pl.BlockSpec((1, BLK_S, 1), lambda b_, i: (b_, i, 0)),
            pl.BlockSpec((1, BLK_S, E), lambda b_, i: (b_, i, 0)),
            pl.BlockSpec((1, 1), lambda b_, i: (0, 0)),
        ],
        out_shape=[
            jax.ShapeDtypeStruct((B, S, E), jnp.int32),
            jax.ShapeDtypeStruct((B, S, 1), jnp.float32),
            jax.ShapeDtypeStruct((B, S, E), jnp.float32),
            jax.ShapeDtypeStruct((1, 1), jnp.float32),
        ],
        scratch_shapes=[
            pltpu.VMEM((1, E), jnp.float32),
            pltpu.VMEM((1, E), jnp.float32),
            pltpu.VMEM((1, E), jnp.float32),
        ],
        compiler_params=pltpu.CompilerParams(
            dimension_semantics=("arbitrary", "arbitrary")),
    )(hidden_states, W, b.reshape(1, E))

    return (ei, tp, rp, aux[0, 0])


# speedup vs baseline: 1.2884x; 1.2884x over previous
import functools

import jax
import jax.numpy as jnp
from jax.experimental import pallas as pl
from jax.experimental.pallas import tpu as pltpu

BATCH = 4
SEQ_LEN = 8192
D_MODEL = 4096
N_EXPERTS = 64
EXPERT_CAPACITY = 160

BLK_S = 1024


def _router_block(x_ref, w_ref, b_ref, ei_ref, tp_ref, rp_ref, aux_ref,
                  carry_ref, fi_ref, pi_ref):
    b = pl.program_id(0)
    i = pl.program_id(1)
    nblk = pl.num_programs(1)

    @pl.when(i == 0)
    def _reset():
        carry_ref[...] = jnp.zeros_like(carry_ref)
        fi_ref[...] = jnp.zeros_like(fi_ref)
        pi_ref[...] = jnp.zeros_like(pi_ref)

    x = x_ref[0]
    logits = jnp.dot(x, w_ref[...],
                     preferred_element_type=jnp.float32) + b_ref[...]
    m = jnp.max(logits, axis=-1, keepdims=True)
    e = jnp.exp(logits - m)
    s = jnp.sum(e, axis=-1, keepdims=True)
    probs = e / s
    rp_ref[0] = probs

    maxp = jnp.max(probs, axis=-1, keepdims=True)
    tp_ref[0] = maxp

    lane = jax.lax.broadcasted_iota(jnp.int32, probs.shape, 1)
    cand = jnp.where(probs >= maxp, lane, N_EXPERTS)
    top_idx = jnp.min(cand, axis=-1, keepdims=True)
    onehot_f = (lane == top_idx).astype(jnp.float32)

    row = jax.lax.broadcasted_iota(jnp.int32, (BLK_S, BLK_S), 0)
    col = jax.lax.broadcasted_iota(jnp.int32, (BLK_S, BLK_S), 1)
    tri = (row >= col).astype(jnp.float32)
    prio_local = jax.lax.dot_general(
        tri, onehot_f, (((1,), (0,)), ((), ())),
        preferred_element_type=jnp.float32)
    prio = prio_local + carry_ref[...]
    keep = prio <= EXPERT_CAPACITY
    kept = jnp.where(keep, onehot_f, 0.0)
    ei_ref[0] = kept.astype(jnp.int32)

    carry_ref[...] = prio[BLK_S - 1:BLK_S, :]
    fi_ref[...] += jnp.sum(kept, axis=0, keepdims=True)
    pi_ref[...] += jnp.sum(probs, axis=0, keepdims=True)

    @pl.when(i == nblk - 1)
    def _aux():
        partial = (N_EXPERTS / (BATCH * float(SEQ_LEN) * float(SEQ_LEN))) * \
            jnp.sum(fi_ref[...] * pi_ref[...])

        @pl.when(b == 0)
        def _init():
            aux_ref[...] = jnp.full((1, 1), partial, jnp.float32)

        @pl.when(b != 0)
        def _acc():
            aux_ref[...] += partial


@jax.jit
def kernel(hidden_states, W, b):
    B, S, D = hidden_states.shape
    E = W.shape[1]
    nblk = S // BLK_S
    grid = (B, nblk)

    ei, tp, rp, aux = pl.pallas_call(
        _router_block,
        grid=grid,
        in_specs=[
            pl.BlockSpec((1, BLK_S, D), lambda b_, i: (b_, i, 0)),
            pl.BlockSpec((D, E), lambda b_, i: (0, 0)),
            pl.BlockSpec((1, E), lambda b_, i: (0, 0)),
        ],
        out_specs=[
            pl.BlockSpec((1, BLK_S, E), lambda b_, i: (b_, i, 0)),
            pl.BlockSpec((1, BLK_S, 1), lambda b_, i: (b_, i, 0)),
            pl.BlockSpec((1, BLK_S, E), lambda b_, i: (b_, i, 0)),
            pl.BlockSpec((1, 1), lambda b_, i: (0, 0)),
        ],
        out_shape=[
            jax.ShapeDtypeStruct((B, S, E), jnp.int32),
            jax.ShapeDtypeStruct((B, S, 1), jnp.float32),
            jax.ShapeDtypeStruct((B, S, E), jnp.float32),
            jax.ShapeDtypeStruct((1, 1), jnp.float32),
        ],
        scratch_shapes=[
            pltpu.VMEM((1, E), jnp.float32),
            pltpu.VMEM((1, E), jnp.float32),
            pltpu.VMEM((1, E), jnp.float32),
        ],
        compiler_params=pltpu.CompilerParams(
            dimension_semantics=("arbitrary", "arbitrary")),
    )(hidden_states, W, b.reshape(1, E))

    return (ei, tp, rp, aux[0, 0])


# speedup vs baseline: 1.3335x; 1.0350x over previous
import functools

import jax
import jax.numpy as jnp
from jax.experimental import pallas as pl
from jax.experimental.pallas import tpu as pltpu

BATCH = 4
SEQ_LEN = 8192
D_MODEL = 4096
N_EXPERTS = 64
EXPERT_CAPACITY = 160

BLK_S = 1024


def _router_block(x_ref, w_ref, b_ref, ei_ref, tp_ref, rp_ref, aux_ref,
                  carry_ref, fi_ref, pi_ref):
    b = pl.program_id(0)
    i = pl.program_id(1)
    nblk = pl.num_programs(1)

    @pl.when(i == 0)
    def _reset():
        carry_ref[...] = jnp.zeros_like(carry_ref)
        fi_ref[...] = jnp.zeros_like(fi_ref)
        pi_ref[...] = jnp.zeros_like(pi_ref)

    x = x_ref[0]
    logits = jnp.dot(x, w_ref[...],
                     preferred_element_type=jnp.float32) + b_ref[...]
    m = jnp.max(logits, axis=-1, keepdims=True)
    e = jnp.exp(logits - m)
    s = jnp.sum(e, axis=-1, keepdims=True)
    probs = e / s
    rp_ref[0] = probs

    maxp = jnp.max(probs, axis=-1, keepdims=True)
    tp_ref[0] = maxp

    lane = jax.lax.broadcasted_iota(jnp.int32, probs.shape, 1)
    cand = jnp.where(probs >= maxp, lane, N_EXPERTS)
    top_idx = jnp.min(cand, axis=-1, keepdims=True)
    onehot_f = (lane == top_idx).astype(jnp.float32)

    G = BLK_S // 8
    E = onehot_f.shape[1]
    grp = jnp.sum(onehot_f.reshape(G, 8, E), axis=1)
    rowg = jax.lax.broadcasted_iota(jnp.int32, (G, G), 0)
    colg = jax.lax.broadcasted_iota(jnp.int32, (G, G), 1)
    tri_strict = (rowg > colg).astype(jnp.float32)
    excl = jax.lax.dot_general(
        tri_strict, grp, (((1,), (0,)), ((), ())),
        preferred_element_type=jnp.float32)
    seed = excl + carry_ref[...]
    seed_rows = jnp.pad(seed[:, None, :],
                        ((0, 0), (0, 7), (0, 0))).reshape(BLK_S, E)
    y = onehot_f + seed_rows
    rowmod = jax.lax.broadcasted_iota(jnp.int32, (BLK_S, 1), 0) % 8
    for k in (1, 2, 4):
        y = y + jnp.where(rowmod >= k, jnp.roll(y, k, axis=0), 0.0)
    prio = y
    keep = prio <= EXPERT_CAPACITY
    kept = jnp.where(keep, onehot_f, 0.0)
    ei_ref[0] = kept.astype(jnp.int32)

    carry_ref[...] = prio[BLK_S - 1:BLK_S, :]
    fi_ref[...] += jnp.sum(kept, axis=0, keepdims=True)
    pi_ref[...] += jnp.sum(probs, axis=0, keepdims=True)

    @pl.when(i == nblk - 1)
    def _aux():
        partial = (N_EXPERTS / (BATCH * float(SEQ_LEN) * float(SEQ_LEN))) * \
            jnp.sum(fi_ref[...] * pi_ref[...])

        @pl.when(b == 0)
        def _init():
            aux_ref[...] = jnp.full((1, 1), partial, jnp.float32)

        @pl.when(b != 0)
        def _acc():
            aux_ref[...] += partial


@jax.jit
def kernel(hidden_states, W, b):
    B, S, D = hidden_states.shape
    E = W.shape[1]
    nblk = S // BLK_S
    grid = (B, nblk)

    ei, tp, rp, aux = pl.pallas_call(
        _router_block,
        grid=grid,
        in_specs=[
            pl.BlockSpec((1, BLK_S, D), lambda b_, i: (b_, i, 0)),
            pl.BlockSpec((D, E), lambda b_, i: (0, 0)),
            pl.BlockSpec((1, E), lambda b_, i: (0, 0)),
        ],
        out_specs=[
            pl.BlockSpec((1, BLK_S, E), lambda b_, i: (b_, i, 0)),
            pl.BlockSpec((1, BLK_S, 1), lambda b_, i: (b_, i, 0)),
            pl.BlockSpec((1, BLK_S, E), lambda b_, i: (b_, i, 0)),
            pl.BlockSpec((1, 1), lambda b_, i: (0, 0)),
        ],
        out_shape=[
            jax.ShapeDtypeStruct((B, S, E), jnp.int32),
            jax.ShapeDtypeStruct((B, S, 1), jnp.float32),
            jax.ShapeDtypeStruct((B, S, E), jnp.float32),
            jax.ShapeDtypeStruct((1, 1), jnp.float32),
        ],
        scratch_shapes=[
            pltpu.VMEM((1, E), jnp.float32),
            pltpu.VMEM((1, E), jnp.float32),
            pltpu.VMEM((1, E), jnp.float32),
        ],
        compiler_params=pltpu.CompilerParams(
            dimension_semantics=("arbitrary", "arbitrary")),
    )(hidden_states, W, b.reshape(1, E))

    return (ei, tp, rp, aux[0, 0])
